# Initial kernel scaffold; baseline (speedup 1.0000x reference)
import functools

import jax
import jax.numpy as jnp
from jax import lax
from jax.experimental import pallas as pl
from jax.experimental.pallas import tpu as pltpu
from jax.experimental.pallas import tpu_sc as plsc

f32 = jnp.float32
i32 = jnp.int32

N = 10000
E = 320000
H = 32
D_E = 16

NC = 2
NS = 16
NW = NC * NS
EW = E // NW
K = 2000
NCHUNK = EW // K
NVEC = K // 16
NPT = N // NS


def _node_dense_body(x_ref, w_ref, avs_ref, avd_ref,
                     h_ref, asrc_ref, adst_ref, ms_ref, md_ref):
    h = jnp.dot(x_ref[...], w_ref[...], preferred_element_type=f32)
    h_ref[...] = h
    asrc = jnp.dot(h, avs_ref[...], preferred_element_type=f32)
    adst = jnp.dot(h, avd_ref[...], preferred_element_type=f32)
    asrc_ref[...] = asrc
    adst_ref[...] = adst
    ms_ref[...] = jnp.max(asrc).reshape(1, 1)
    md_ref[...] = jnp.max(adst).reshape(1, 1)


def _node_dense(x, W, a_s, a_d):
    n = x.shape[0]
    return pl.pallas_call(
        _node_dense_body,
        out_shape=(jax.ShapeDtypeStruct((n, H), f32),
                   jax.ShapeDtypeStruct((n, 1), f32),
                   jax.ShapeDtypeStruct((n, 1), f32),
                   jax.ShapeDtypeStruct((1, 1), f32),
                   jax.ShapeDtypeStruct((1, 1), f32)),
    )(x, W, a_s.reshape(H, 1), a_d.reshape(H, 1))


_BE = 20000


def _edge_dense_body(attr_ref, ve_ref, o1_ref, o2_ref, m1_ref, m2_ref):
    pi = pl.program_id(0)
    r = jnp.dot(attr_ref[...], ve_ref[...], preferred_element_type=f32)
    o1_ref[...] = r[:, 0:1]
    o2_ref[...] = r[:, 1:2]

    @pl.when(pi == 0)
    def _():
        m1_ref[...] = jnp.full((1, 1), -jnp.inf, f32)
        m2_ref[...] = jnp.full((1, 1), -jnp.inf, f32)

    m1_ref[...] = jnp.maximum(m1_ref[...], jnp.max(r[:, 0]))
    m2_ref[...] = jnp.maximum(m2_ref[...], jnp.max(r[:, 1]))


def _edge_dense(edge_attr, ve12):
    grid = E // _BE
    return pl.pallas_call(
        _edge_dense_body,
        grid=(grid,),
        in_specs=[pl.BlockSpec((_BE, D_E), lambda i: (i, 0)),
                  pl.BlockSpec((D_E, 2), lambda i: (0, 0))],
        out_specs=(pl.BlockSpec((_BE, 1), lambda i: (i, 0)),
                   pl.BlockSpec((_BE, 1), lambda i: (i, 0)),
                   pl.BlockSpec((1, 1), lambda i: (0, 0)),
                   pl.BlockSpec((1, 1), lambda i: (0, 0))),
        out_shape=(jax.ShapeDtypeStruct((E, 1), f32),
                   jax.ShapeDtypeStruct((E, 1), f32),
                   jax.ShapeDtypeStruct((1, 1), f32),
                   jax.ShapeDtypeStruct((1, 1), f32)),
    )(edge_attr, ve12)


_NB = 2000


def _mid_body(h1_ref, acc32_ref, accp_ref, acce1_ref, acce2_ref, accdeg_ref,
              asrc_ref, adst_ref, bound_ref, b1_ref, w2_ref, as2_ref, ad2_ref,
              h2_ref, asrc2_ref, adst2_ref, aloop2_ref, ms2_ref, md2_ref):
    pi = pl.program_id(0)
    deg = accdeg_ref[0] + accdeg_ref[1]
    degc = jnp.maximum(deg, 1.0)
    aloop1 = (acce1_ref[0] + acce1_ref[1]) / degc
    al = asrc_ref[...] + adst_ref[...] + aloop1
    al = jnp.where(al >= 0, al, 0.2 * al)
    p_loop = jnp.exp(al - bound_ref[0, 0])
    asum = accp_ref[0] + accp_ref[1] + p_loop
    inv = 1.0 / asum
    h1 = h1_ref[...]
    out1 = (acc32_ref[0] + acc32_ref[1] + h1 * p_loop) * inv + b1_ref[...]
    x2 = jnp.maximum(out1, 0.0)
    h2 = jnp.dot(x2, w2_ref[...], preferred_element_type=f32)
    h2_ref[...] = h2
    asrc2 = jnp.dot(h2, as2_ref[...], preferred_element_type=f32)
    adst2 = jnp.dot(h2, ad2_ref[...], preferred_element_type=f32)
    asrc2_ref[...] = asrc2
    adst2_ref[...] = adst2
    aloop2_ref[...] = (acce2_ref[0] + acce2_ref[1]) / degc

    @pl.when(pi == 0)
    def _():
        ms2_ref[...] = jnp.full((1, 1), -jnp.inf, f32)
        md2_ref[...] = jnp.full((1, 1), -jnp.inf, f32)

    ms2_ref[...] = jnp.maximum(ms2_ref[...], jnp.max(asrc2))
    md2_ref[...] = jnp.maximum(md2_ref[...], jnp.max(adst2))


def _mid_dense(h1, acc32, accp, acce1, acce2, accdeg, asrc, adst, bound,
               b1, W2, as2, ad2):
    grid = N // _NB
    col = lambda i: (0, i, 0)
    return pl.pallas_call(
        _mid_body,
        grid=(grid,),
        in_specs=[pl.BlockSpec((_NB, H), lambda i: (i, 0)),
                  pl.BlockSpec((NC, _NB, H), col),
                  pl.BlockSpec((NC, _NB, 1), col),
                  pl.BlockSpec((NC, _NB, 1), col),
                  pl.BlockSpec((NC, _NB, 1), col),
                  pl.BlockSpec((NC, _NB, 1), col),
                  pl.BlockSpec((_NB, 1), lambda i: (i, 0)),
                  pl.BlockSpec((_NB, 1), lambda i: (i, 0)),
                  pl.BlockSpec((1, 1), lambda i: (0, 0)),
                  pl.BlockSpec((1, H), lambda i: (0, 0)),
                  pl.BlockSpec((H, H), lambda i: (0, 0)),
                  pl.BlockSpec((H, 1), lambda i: (0, 0)),
                  pl.BlockSpec((H, 1), lambda i: (0, 0))],
        out_specs=(pl.BlockSpec((_NB, H), lambda i: (i, 0)),
                   pl.BlockSpec((_NB, 1), lambda i: (i, 0)),
                   pl.BlockSpec((_NB, 1), lambda i: (i, 0)),
                   pl.BlockSpec((_NB, 1), lambda i: (i, 0)),
                   pl.BlockSpec((1, 1), lambda i: (0, 0)),
                   pl.BlockSpec((1, 1), lambda i: (0, 0))),
        out_shape=(jax.ShapeDtypeStruct((N, H), f32),
                   jax.ShapeDtypeStruct((N, 1), f32),
                   jax.ShapeDtypeStruct((N, 1), f32),
                   jax.ShapeDtypeStruct((N, 1), f32),
                   jax.ShapeDtypeStruct((1, 1), f32),
                   jax.ShapeDtypeStruct((1, 1), f32)),
    )(h1, acc32, accp, acce1, acce2, accdeg, asrc, adst, bound,
      b1.reshape(1, H), W2, as2.reshape(H, 1), ad2.reshape(H, 1))


def _final_body(h2_ref, acc32_ref, accp_ref, asrc_ref, adst_ref, aloop_ref,
                bound_ref, b2_ref, wl_ref, bl_ref, out_ref):
    al = asrc_ref[...] + adst_ref[...] + aloop_ref[...]
    al = jnp.where(al >= 0, al, 0.2 * al)
    p_loop = jnp.exp(al - bound_ref[0, 0])
    asum = accp_ref[0] + accp_ref[1] + p_loop
    inv = 1.0 / asum
    h2 = h2_ref[...]
    out2 = (acc32_ref[0] + acc32_ref[1] + h2 * p_loop) * inv + b2_ref[...]
    o = jnp.dot(out2, wl_ref[...], preferred_element_type=f32) + bl_ref[...]
    out_ref[...] = jnp.maximum(o, 0.0)


def _final_dense(h2, acc32, accp, asrc2, adst2, aloop2, bound2, b2, Wl, bl):
    grid = N // _NB
    col = lambda i: (0, i, 0)
    return pl.pallas_call(
        _final_body,
        grid=(grid,),
        in_specs=[pl.BlockSpec((_NB, H), lambda i: (i, 0)),
                  pl.BlockSpec((NC, _NB, H), col),
                  pl.BlockSpec((NC, _NB, 1), col),
                  pl.BlockSpec((_NB, 1), lambda i: (i, 0)),
                  pl.BlockSpec((_NB, 1), lambda i: (i, 0)),
                  pl.BlockSpec((_NB, 1), lambda i: (i, 0)),
                  pl.BlockSpec((1, 1), lambda i: (0, 0)),
                  pl.BlockSpec((1, H), lambda i: (0, 0)),
                  pl.BlockSpec((H, 1), lambda i: (0, 0)),
                  pl.BlockSpec((1, 1), lambda i: (0, 0))],
        out_specs=pl.BlockSpec((_NB, 1), lambda i: (i, 0)),
        out_shape=jax.ShapeDtypeStruct((N, 1), f32),
    )(h2, acc32, accp, asrc2, adst2, aloop2, bound2,
      b2.reshape(1, H), Wl, bl.reshape(1, 1))


_MESH = plsc.VectorSubcoreMesh(core_axis_name="c", subcore_axis_name="s")


def _pass1_body_l1(src_hbm, dst_hbm, ae1_hbm, ae2_hbm, asrc_hbm, adst_hbm,
                   bound_hbm, p_hbm, accp_hbm, acce1_hbm, acce2_hbm,
                   accdeg_hbm, asrc_v, adst_v, bound_v, src_v, dst_v, ae1_v,
                   ae2_v, ones_v, p_v, accp_sh, acce1_sh, acce2_sh, accdeg_sh):
    _pass1_common(src_hbm, dst_hbm, ae1_hbm, asrc_hbm, adst_hbm, bound_hbm,
                  p_hbm, accp_hbm, asrc_v, adst_v, bound_v, src_v, dst_v,
                  ae1_v, p_v, accp_sh,
                  extras=(ae2_hbm, acce1_hbm, acce2_hbm, accdeg_hbm,
                          ae2_v, ones_v, acce1_sh, acce2_sh, accdeg_sh))


def _pass1_body_l2(src_hbm, dst_hbm, ae_hbm, asrc_hbm, adst_hbm, bound_hbm,
                   p_hbm, accp_hbm, asrc_v, adst_v, bound_v, src_v, dst_v,
                   ae_v, p_v, accp_sh):
    _pass1_common(src_hbm, dst_hbm, ae_hbm, asrc_hbm, adst_hbm, bound_hbm,
                  p_hbm, accp_hbm, asrc_v, adst_v, bound_v, src_v, dst_v,
                  ae_v, p_v, accp_sh, extras=None)


def _pass1_common(src_hbm, dst_hbm, ae_hbm, asrc_hbm, adst_hbm, bound_hbm,
                  p_hbm, accp_hbm, asrc_v, adst_v, bound_v, src_v, dst_v,
                  ae_v, p_v, accp_sh, extras):
    cid = lax.axis_index("c")
    sid = lax.axis_index("s")
    wid = sid * NC + cid

    pltpu.sync_copy(asrc_hbm, asrc_v)
    pltpu.sync_copy(adst_hbm, adst_v)
    pltpu.sync_copy(bound_hbm, bound_v)

    if extras is not None:
        (ae2_hbm, acce1_hbm, acce2_hbm, accdeg_hbm,
         ae2_v, ones_v, acce1_sh, acce2_sh, accdeg_sh) = extras
        one = jnp.ones((16,), f32)

        def fill_ones(i, _):
            ones_v[pl.ds(i * 16, 16)] = one
            return 0

        lax.fori_loop(0, NVEC, fill_ones, 0)

    z = jnp.zeros((16,), f32)

    def fill_zero(i, _):
        p_v[pl.ds(i * 16, 16)] = z
        return 0

    lax.fori_loop(0, NVEC, fill_zero, 0)

    @pl.when(sid == 0)
    def _():
        for j in range(N // K):
            pltpu.sync_copy(p_v, accp_sh.at[pl.ds(j * K, K)])
            if extras is not None:
                pltpu.sync_copy(p_v, acce1_sh.at[pl.ds(j * K, K)])
                pltpu.sync_copy(p_v, acce2_sh.at[pl.ds(j * K, K)])
                pltpu.sync_copy(p_v, accdeg_sh.at[pl.ds(j * K, K)])

    plsc.subcore_barrier()

    bound = bound_v[...]

    for j in range(NCHUNK):
        base = wid * EW + j * K
        pltpu.sync_copy(src_hbm.at[pl.ds(base, K)], src_v)
        pltpu.sync_copy(dst_hbm.at[pl.ds(base, K)], dst_v)
        pltpu.sync_copy(ae_hbm.at[pl.ds(base, K)], ae_v)
        if extras is not None:
            pltpu.sync_copy(ae2_hbm.at[pl.ds(base, K)], ae2_v)

        def vec_body(i, _):
            sv = src_v[pl.ds(i * 16, 16)]
            dv = dst_v[pl.ds(i * 16, 16)]
            a = (plsc.load_gather(asrc_v, [sv])
                 + plsc.load_gather(adst_v, [dv])
                 + ae_v[pl.ds(i * 16, 16)])
            a = jnp.where(a >= 0, a, 0.2 * a)
            p_v[pl.ds(i * 16, 16)] = jnp.exp(a - bound)
            return 0

        lax.fori_loop(0, NVEC, vec_body, 0)

        pltpu.sync_copy(p_v, p_hbm.at[pl.ds(base, K)])
        pltpu.sync_copy(p_v, accp_sh.at[dst_v], add=True)
        if extras is not None:
            pltpu.sync_copy(ae_v, acce1_sh.at[dst_v], add=True)
            pltpu.sync_copy(ae2_v, acce2_sh.at[dst_v], add=True)
            pltpu.sync_copy(ones_v, accdeg_sh.at[dst_v], add=True)

    plsc.subcore_barrier()

    @pl.when(sid == 0)
    def _():
        pltpu.sync_copy(accp_sh, accp_hbm.at[cid])
        if extras is not None:
            pltpu.sync_copy(acce1_sh, acce1_hbm.at[cid])
            pltpu.sync_copy(acce2_sh, acce2_hbm.at[cid])
            pltpu.sync_copy(accdeg_sh, accdeg_hbm.at[cid])


def _make_pass1_l1():
    scratch = [
        pltpu.VMEM((N,), f32),
        pltpu.VMEM((N,), f32),
        pltpu.VMEM((16,), f32),
        pltpu.VMEM((K,), i32),
        pltpu.VMEM((K,), i32),
        pltpu.VMEM((K,), f32),
        pltpu.VMEM((K,), f32),
        pltpu.VMEM((K,), f32),
        pltpu.VMEM((K,), f32),
        pltpu.VMEM_SHARED((N,), f32),
        pltpu.VMEM_SHARED((N,), f32),
        pltpu.VMEM_SHARED((N,), f32),
        pltpu.VMEM_SHARED((N,), f32),
    ]
    out_type = (jax.ShapeDtypeStruct((E,), f32),
                jax.ShapeDtypeStruct((NC, N), f32),
                jax.ShapeDtypeStruct((NC, N), f32),
                jax.ShapeDtypeStruct((NC, N), f32),
                jax.ShapeDtypeStruct((NC, N), f32))
    return pl.kernel(_pass1_body_l1, out_type=out_type, mesh=_MESH,
                     scratch_types=scratch)


def _make_pass1_l2():
    scratch = [
        pltpu.VMEM((N,), f32),
        pltpu.VMEM((N,), f32),
        pltpu.VMEM((16,), f32),
        pltpu.VMEM((K,), i32),
        pltpu.VMEM((K,), i32),
        pltpu.VMEM((K,), f32),
        pltpu.VMEM((K,), f32),
        pltpu.VMEM_SHARED((N,), f32),
    ]
    out_type = (jax.ShapeDtypeStruct((E,), f32),
                jax.ShapeDtypeStruct((NC, N), f32))
    return pl.kernel(_pass1_body_l2, out_type=out_type, mesh=_MESH,
                     scratch_types=scratch)


def _pass2_body(src_hbm, dst_hbm, p_hbm, h_hbm, acc_hbm,
                src_v, dst_v, p_v, rows_v, zrow_v, acc_sh):
    cid = lax.axis_index("c")
    sid = lax.axis_index("s")
    wid = sid * NC + cid

    z = jnp.zeros((16,), f32)

    def zbody(i, _):
        zrow_v[i, pl.ds(0, 16)] = z
        zrow_v[i, pl.ds(16, 16)] = z
        return 0

    lax.fori_loop(0, NPT, zbody, 0)
    pltpu.sync_copy(zrow_v, acc_sh.at[pl.ds(sid * NPT, NPT)])
    plsc.subcore_barrier()

    for j in range(NCHUNK):
        base = wid * EW + j * K
        pltpu.sync_copy(src_hbm.at[pl.ds(base, K)], src_v)
        pltpu.sync_copy(dst_hbm.at[pl.ds(base, K)], dst_v)
        pltpu.sync_copy(p_hbm.at[pl.ds(base, K)], p_v)
        pltpu.sync_copy(h_hbm.at[src_v], rows_v)

        def row_body(i, _):
            b = jnp.full((16,), p_v[i], f32)
            rows_v[i, pl.ds(0, 16)] = rows_v[i, pl.ds(0, 16)] * b
            rows_v[i, pl.ds(16, 16)] = rows_v[i, pl.ds(16, 16)] * b
            return 0

        lax.fori_loop(0, K, row_body, 0)

        pltpu.sync_copy(rows_v, acc_sh.at[dst_v], add=True)

    plsc.subcore_barrier()
    pltpu.sync_copy(acc_sh.at[pl.ds(sid * NPT, NPT)],
                    acc_hbm.at[cid, pl.ds(sid * NPT, NPT)])


def _make_pass2():
    scratch = [
        pltpu.VMEM((K,), i32),
        pltpu.VMEM((K,), i32),
        pltpu.VMEM((K,), f32),
        pltpu.VMEM((K, H), f32),
        pltpu.VMEM((NPT, H), f32),
        pltpu.VMEM_SHARED((N, H), f32),
    ]
    out_type = jax.ShapeDtypeStruct((NC, N, H), f32)
    return pl.kernel(_pass2_body, out_type=out_type, mesh=_MESH,
                     scratch_types=scratch)


def kernel(x, edge_index, edge_attr, W1, as1, ad1, We1, ae1, b1,
           W2, as2, ad2, We2, ae2, b2, Wl, bl):
    src = edge_index[0]
    dst = edge_index[1]

    ve12 = jnp.stack([We1 @ ae1, We2 @ ae2], axis=1)

    h1, asrc1, adst1, ms1, md1 = _node_dense(x, W1, as1, ad1)
    ae_e1, ae_e2, mae1, mae2 = _edge_dense(edge_attr, ve12)

    bound1 = ms1[0, 0] + md1[0, 0] + jnp.maximum(mae1[0, 0], 0.0)
    bound1_v = jnp.full((16,), bound1, f32)
    bound1_s = bound1.reshape(1, 1)

    p1, accp1, acce1, acce2, accdeg = _make_pass1_l1()(
        src, dst, ae_e1.reshape(E), ae_e2.reshape(E),
        asrc1.reshape(N), adst1.reshape(N), bound1_v)

    acc32_1 = _make_pass2()(src, dst, p1, h1)

    h2, asrc2, adst2, aloop2, ms2, md2 = _mid_dense(
        h1, acc32_1, accp1.reshape(NC, N, 1), acce1.reshape(NC, N, 1),
        acce2.reshape(NC, N, 1), accdeg.reshape(NC, N, 1),
        asrc1, adst1, bound1_s, b1, W2, as2, ad2)

    bound2 = ms2[0, 0] + md2[0, 0] + jnp.maximum(mae2[0, 0], 0.0)
    bound2_v = jnp.full((16,), bound2, f32)
    bound2_s = bound2.reshape(1, 1)

    p2, accp2 = _make_pass1_l2()(
        src, dst, ae_e2.reshape(E),
        asrc2.reshape(N), adst2.reshape(N), bound2_v)

    acc32_2 = _make_pass2()(src, dst, p2, h2)

    out = _final_dense(h2, acc32_2, accp2.reshape(NC, N, 1),
                       asrc2, adst2, aloop2, bound2_s, b2, Wl, bl)
    return out


# speedup vs baseline: 30.7078x; 30.7078x over previous
import functools

import jax
import jax.numpy as jnp
from jax import lax
from jax.experimental import pallas as pl
from jax.experimental.pallas import tpu as pltpu
from jax.experimental.pallas import tpu_sc as plsc

f32 = jnp.float32
i32 = jnp.int32

N = 10000
E = 320000
H = 32
D_E = 16

NC = 2
NS = 16
NW = NC * NS
EW = E // NW
K = 2000
NCHUNK = EW // K
NVEC = K // 16
ZR = 1000


def _node_dense_body(x_ref, w_ref, avs_ref, avd_ref,
                     h_ref, asrc_ref, adst_ref, ms_ref, md_ref):
    h = jnp.dot(x_ref[...], w_ref[...], preferred_element_type=f32)
    h_ref[...] = h
    asrc = jnp.dot(h, avs_ref[...], preferred_element_type=f32)
    adst = jnp.dot(h, avd_ref[...], preferred_element_type=f32)
    asrc_ref[...] = asrc
    adst_ref[...] = adst
    ms_ref[...] = jnp.max(asrc).reshape(1, 1)
    md_ref[...] = jnp.max(adst).reshape(1, 1)


def _node_dense(x, W, a_s, a_d):
    n = x.shape[0]
    return pl.pallas_call(
        _node_dense_body,
        out_shape=(jax.ShapeDtypeStruct((n, H), f32),
                   jax.ShapeDtypeStruct((n, 1), f32),
                   jax.ShapeDtypeStruct((n, 1), f32),
                   jax.ShapeDtypeStruct((1, 1), f32),
                   jax.ShapeDtypeStruct((1, 1), f32)),
    )(x, W, a_s.reshape(H, 1), a_d.reshape(H, 1))


_BE = 4000


def _edge_dense_body(attr_ref, ve_ref, o1_ref, o2_ref, m1_ref, m2_ref):
    pi = pl.program_id(0)
    r = jnp.dot(attr_ref[...], ve_ref[...], preferred_element_type=f32)
    o1_ref[...] = r[:, 0:1]
    o2_ref[...] = r[:, 1:2]

    @pl.when(pi == 0)
    def _():
        m1_ref[...] = jnp.full((1, 1), -jnp.inf, f32)
        m2_ref[...] = jnp.full((1, 1), -jnp.inf, f32)

    m1_ref[...] = jnp.maximum(m1_ref[...], jnp.max(r[:, 0]))
    m2_ref[...] = jnp.maximum(m2_ref[...], jnp.max(r[:, 1]))


def _edge_dense(edge_attr, ve12):
    grid = E // _BE
    return pl.pallas_call(
        _edge_dense_body,
        grid=(grid,),
        in_specs=[pl.BlockSpec((_BE, D_E), lambda i: (i, 0)),
                  pl.BlockSpec((D_E, 2), lambda i: (0, 0))],
        out_specs=(pl.BlockSpec((_BE, 1), lambda i: (i, 0)),
                   pl.BlockSpec((_BE, 1), lambda i: (i, 0)),
                   pl.BlockSpec((1, 1), lambda i: (0, 0)),
                   pl.BlockSpec((1, 1), lambda i: (0, 0))),
        out_shape=(jax.ShapeDtypeStruct((E, 1), f32),
                   jax.ShapeDtypeStruct((E, 1), f32),
                   jax.ShapeDtypeStruct((1, 1), f32),
                   jax.ShapeDtypeStruct((1, 1), f32)),
    )(edge_attr, ve12)


_NB = 1000


def _mid_body(h1_ref, acc32_ref, accp_ref, acce1_ref, acce2_ref, accdeg_ref,
              asrc_ref, adst_ref, bound_ref, b1_ref, w2_ref, as2_ref, ad2_ref,
              h2_ref, asrc2_ref, adst2_ref, aloop2_ref, ms2_ref, md2_ref):
    pi = pl.program_id(0)
    deg = accdeg_ref[0] + accdeg_ref[1]
    degc = jnp.maximum(deg, 1.0)
    aloop1 = (acce1_ref[0] + acce1_ref[1]) / degc
    al = asrc_ref[...] + adst_ref[...] + aloop1
    al = jnp.where(al >= 0, al, 0.2 * al)
    p_loop = jnp.exp(al - bound_ref[0, 0])
    asum = accp_ref[0] + accp_ref[1] + p_loop
    inv = 1.0 / asum
    h1 = h1_ref[...]
    out1 = (acc32_ref[0] + acc32_ref[1] + h1 * p_loop) * inv + b1_ref[...]
    x2 = jnp.maximum(out1, 0.0)
    h2 = jnp.dot(x2, w2_ref[...], preferred_element_type=f32)
    h2_ref[...] = h2
    asrc2 = jnp.dot(h2, as2_ref[...], preferred_element_type=f32)
    adst2 = jnp.dot(h2, ad2_ref[...], preferred_element_type=f32)
    asrc2_ref[...] = asrc2
    adst2_ref[...] = adst2
    aloop2_ref[...] = (acce2_ref[0] + acce2_ref[1]) / degc

    @pl.when(pi == 0)
    def _():
        ms2_ref[...] = jnp.full((1, 1), -jnp.inf, f32)
        md2_ref[...] = jnp.full((1, 1), -jnp.inf, f32)

    ms2_ref[...] = jnp.maximum(ms2_ref[...], jnp.max(asrc2))
    md2_ref[...] = jnp.maximum(md2_ref[...], jnp.max(adst2))


def _mid_dense(h1, acc32, accp, acce1, acce2, accdeg, asrc, adst, bound,
               b1, W2, as2, ad2):
    grid = N // _NB
    col = lambda i: (0, i, 0)
    return pl.pallas_call(
        _mid_body,
        grid=(grid,),
        in_specs=[pl.BlockSpec((_NB, H), lambda i: (i, 0)),
                  pl.BlockSpec((NC, _NB, H), col),
                  pl.BlockSpec((NC, _NB, 1), col),
                  pl.BlockSpec((NC, _NB, 1), col),
                  pl.BlockSpec((NC, _NB, 1), col),
                  pl.BlockSpec((NC, _NB, 1), col),
                  pl.BlockSpec((_NB, 1), lambda i: (i, 0)),
                  pl.BlockSpec((_NB, 1), lambda i: (i, 0)),
                  pl.BlockSpec((1, 1), lambda i: (0, 0)),
                  pl.BlockSpec((1, H), lambda i: (0, 0)),
                  pl.BlockSpec((H, H), lambda i: (0, 0)),
                  pl.BlockSpec((H, 1), lambda i: (0, 0)),
                  pl.BlockSpec((H, 1), lambda i: (0, 0))],
        out_specs=(pl.BlockSpec((_NB, H), lambda i: (i, 0)),
                   pl.BlockSpec((_NB, 1), lambda i: (i, 0)),
                   pl.BlockSpec((_NB, 1), lambda i: (i, 0)),
                   pl.BlockSpec((_NB, 1), lambda i: (i, 0)),
                   pl.BlockSpec((1, 1), lambda i: (0, 0)),
                   pl.BlockSpec((1, 1), lambda i: (0, 0))),
        out_shape=(jax.ShapeDtypeStruct((N, H), f32),
                   jax.ShapeDtypeStruct((N, 1), f32),
                   jax.ShapeDtypeStruct((N, 1), f32),
                   jax.ShapeDtypeStruct((N, 1), f32),
                   jax.ShapeDtypeStruct((1, 1), f32),
                   jax.ShapeDtypeStruct((1, 1), f32)),
    )(h1, acc32, accp, acce1, acce2, accdeg, asrc, adst, bound,
      b1.reshape(1, H), W2, as2.reshape(H, 1), ad2.reshape(H, 1))


def _final_body(h2_ref, acc32_ref, accp_ref, asrc_ref, adst_ref, aloop_ref,
                bound_ref, b2_ref, wl_ref, bl_ref, out_ref):
    al = asrc_ref[...] + adst_ref[...] + aloop_ref[...]
    al = jnp.where(al >= 0, al, 0.2 * al)
    p_loop = jnp.exp(al - bound_ref[0, 0])
    asum = accp_ref[0] + accp_ref[1] + p_loop
    inv = 1.0 / asum
    h2 = h2_ref[...]
    out2 = (acc32_ref[0] + acc32_ref[1] + h2 * p_loop) * inv + b2_ref[...]
    o = jnp.dot(out2, wl_ref[...], preferred_element_type=f32) + bl_ref[...]
    out_ref[...] = jnp.maximum(o, 0.0)


def _final_dense(h2, acc32, accp, asrc2, adst2, aloop2, bound2, b2, Wl, bl):
    grid = N // _NB
    col = lambda i: (0, i, 0)
    return pl.pallas_call(
        _final_body,
        grid=(grid,),
        in_specs=[pl.BlockSpec((_NB, H), lambda i: (i, 0)),
                  pl.BlockSpec((NC, _NB, H), col),
                  pl.BlockSpec((NC, _NB, 1), col),
                  pl.BlockSpec((_NB, 1), lambda i: (i, 0)),
                  pl.BlockSpec((_NB, 1), lambda i: (i, 0)),
                  pl.BlockSpec((_NB, 1), lambda i: (i, 0)),
                  pl.BlockSpec((1, 1), lambda i: (0, 0)),
                  pl.BlockSpec((1, H), lambda i: (0, 0)),
                  pl.BlockSpec((H, 1), lambda i: (0, 0)),
                  pl.BlockSpec((1, 1), lambda i: (0, 0))],
        out_specs=pl.BlockSpec((_NB, 1), lambda i: (i, 0)),
        out_shape=jax.ShapeDtypeStruct((N, 1), f32),
    )(h2, acc32, accp, asrc2, adst2, aloop2, bound2,
      b2.reshape(1, H), Wl, bl.reshape(1, 1))


_MESH = plsc.VectorSubcoreMesh(core_axis_name="c", subcore_axis_name="s")
_SC_PARAMS = pltpu.CompilerParams(needs_layout_passes=False)
_SC_PARAMS_NT = pltpu.CompilerParams(needs_layout_passes=False,
                                     use_tc_tiling_on_sc=False)


def _pass1_body_l1(src_hbm, dst_hbm, ae1_hbm, ae2_hbm, asrc_hbm, adst_hbm,
                   bound_hbm, p_hbm, accp_hbm, acce1_hbm, acce2_hbm,
                   accdeg_hbm, asrc_v, adst_v, bound_v, src_v, dst_v, ae1_v,
                   ae2_v, ones_v, p_v, accp_sh, acce1_sh, acce2_sh, accdeg_sh):
    _pass1_common(src_hbm, dst_hbm, ae1_hbm, asrc_hbm, adst_hbm, bound_hbm,
                  p_hbm, accp_hbm, asrc_v, adst_v, bound_v, src_v, dst_v,
                  ae1_v, p_v, accp_sh,
                  extras=(ae2_hbm, acce1_hbm, acce2_hbm, accdeg_hbm,
                          ae2_v, ones_v, acce1_sh, acce2_sh, accdeg_sh))


def _pass1_body_l2(src_hbm, dst_hbm, ae_hbm, asrc_hbm, adst_hbm, bound_hbm,
                   p_hbm, accp_hbm, asrc_v, adst_v, bound_v, src_v, dst_v,
                   ae_v, p_v, accp_sh):
    _pass1_common(src_hbm, dst_hbm, ae_hbm, asrc_hbm, adst_hbm, bound_hbm,
                  p_hbm, accp_hbm, asrc_v, adst_v, bound_v, src_v, dst_v,
                  ae_v, p_v, accp_sh, extras=None)


def _pass1_common(src_hbm, dst_hbm, ae_hbm, asrc_hbm, adst_hbm, bound_hbm,
                  p_hbm, accp_hbm, asrc_v, adst_v, bound_v, src_v, dst_v,
                  ae_v, p_v, accp_sh, extras):
    cid = lax.axis_index("c")
    sid = lax.axis_index("s")
    wid = sid * NC + cid

    pltpu.sync_copy(asrc_hbm, asrc_v)
    pltpu.sync_copy(adst_hbm, adst_v)
    pltpu.sync_copy(bound_hbm, bound_v)

    if extras is not None:
        (ae2_hbm, acce1_hbm, acce2_hbm, accdeg_hbm,
         ae2_v, ones_v, acce1_sh, acce2_sh, accdeg_sh) = extras
        one = jnp.ones((16,), f32)

        def fill_ones(i, _):
            ones_v[pl.ds(i * 16, 16)] = one
            return 0

        lax.fori_loop(0, NVEC, fill_ones, 0)

    z = jnp.zeros((16,), f32)

    def fill_zero(i, _):
        p_v[pl.ds(i * 16, 16)] = z
        return 0

    lax.fori_loop(0, NVEC, fill_zero, 0)

    @pl.when(sid == 0)
    def _():
        for j in range(N // K):
            pltpu.sync_copy(p_v, accp_sh.at[pl.ds(j * K, K)])
            if extras is not None:
                pltpu.sync_copy(p_v, acce1_sh.at[pl.ds(j * K, K)])
                pltpu.sync_copy(p_v, acce2_sh.at[pl.ds(j * K, K)])
                pltpu.sync_copy(p_v, accdeg_sh.at[pl.ds(j * K, K)])

    plsc.subcore_barrier()

    bound = bound_v[...]

    for j in range(NCHUNK):
        base = wid * EW + j * K
        pltpu.sync_copy(src_hbm.at[pl.ds(base, K)], src_v)
        pltpu.sync_copy(dst_hbm.at[pl.ds(base, K)], dst_v)
        pltpu.sync_copy(ae_hbm.at[pl.ds(base, K)], ae_v)
        if extras is not None:
            pltpu.sync_copy(ae2_hbm.at[pl.ds(base, K)], ae2_v)

        def vec_body(i, _):
            sv = src_v[pl.ds(i * 16, 16)]
            dv = dst_v[pl.ds(i * 16, 16)]
            a = (plsc.load_gather(asrc_v, [sv])
                 + plsc.load_gather(adst_v, [dv])
                 + ae_v[pl.ds(i * 16, 16)])
            a = jnp.where(a >= 0, a, 0.2 * a)
            p_v[pl.ds(i * 16, 16)] = jnp.exp(a - bound)
            return 0

        lax.fori_loop(0, NVEC, vec_body, 0)

        pltpu.sync_copy(p_v, p_hbm.at[pl.ds(base, K)])
        pltpu.sync_copy(p_v, accp_sh.at[dst_v], add=True)
        if extras is not None:
            pltpu.sync_copy(ae_v, acce1_sh.at[dst_v], add=True)
            pltpu.sync_copy(ae2_v, acce2_sh.at[dst_v], add=True)
            pltpu.sync_copy(ones_v, accdeg_sh.at[dst_v], add=True)

    plsc.subcore_barrier()

    npairs = N // K

    def dump(sh, hbm, slot):
        @pl.when(sid == slot)
        def _():
            for j in range(npairs):
                pltpu.sync_copy(sh.at[pl.ds(j * K, K)], p_v)
                pltpu.sync_copy(p_v, hbm.at[pl.ds(cid * N + j * K, K)])

    dump(accp_sh, accp_hbm, 0)
    if extras is not None:
        dump(acce1_sh, acce1_hbm, 1)
        dump(acce2_sh, acce2_hbm, 2)
        dump(accdeg_sh, accdeg_hbm, 3)


def _make_pass1_l1():
    scratch = [
        pltpu.VMEM((N,), f32),
        pltpu.VMEM((N,), f32),
        pltpu.VMEM((16,), f32),
        pltpu.VMEM((K,), i32),
        pltpu.VMEM((K,), i32),
        pltpu.VMEM((K,), f32),
        pltpu.VMEM((K,), f32),
        pltpu.VMEM((K,), f32),
        pltpu.VMEM((K,), f32),
        pltpu.VMEM_SHARED((N,), f32),
        pltpu.VMEM_SHARED((N,), f32),
        pltpu.VMEM_SHARED((N,), f32),
        pltpu.VMEM_SHARED((N,), f32),
    ]
    out_type = (jax.ShapeDtypeStruct((E,), f32),
                jax.ShapeDtypeStruct((NC * N,), f32),
                jax.ShapeDtypeStruct((NC * N,), f32),
                jax.ShapeDtypeStruct((NC * N,), f32),
                jax.ShapeDtypeStruct((NC * N,), f32))
    return pl.kernel(_pass1_body_l1, out_type=out_type, mesh=_MESH,
                     scratch_types=scratch, compiler_params=_SC_PARAMS)


def _make_pass1_l2():
    scratch = [
        pltpu.VMEM((N,), f32),
        pltpu.VMEM((N,), f32),
        pltpu.VMEM((16,), f32),
        pltpu.VMEM((K,), i32),
        pltpu.VMEM((K,), i32),
        pltpu.VMEM((K,), f32),
        pltpu.VMEM((K,), f32),
        pltpu.VMEM_SHARED((N,), f32),
    ]
    out_type = (jax.ShapeDtypeStruct((E,), f32),
                jax.ShapeDtypeStruct((NC * N,), f32))
    return pl.kernel(_pass1_body_l2, out_type=out_type, mesh=_MESH,
                     scratch_types=scratch, compiler_params=_SC_PARAMS)


def _pass2_body(src_hbm, dst_hbm, p_hbm, h_hbm, acc_hbm,
                src_v, dst_v, p_v, rows_v, zrow_v, acc_sh):
    cid = lax.axis_index("c")
    sid = lax.axis_index("s")
    wid = sid * NC + cid

    z = jnp.zeros((16,), f32)

    def zbody(i, _):
        zrow_v[i, pl.ds(0, 16)] = z
        zrow_v[i, pl.ds(16, 16)] = z
        return 0

    lax.fori_loop(0, ZR, zbody, 0)

    @pl.when(sid < N // ZR)
    def _():
        pltpu.sync_copy(zrow_v, acc_sh.at[pl.ds(sid * ZR, ZR)])

    plsc.subcore_barrier()

    for j in range(NCHUNK):
        base = wid * EW + j * K
        pltpu.sync_copy(src_hbm.at[pl.ds(base, K)], src_v)
        pltpu.sync_copy(dst_hbm.at[pl.ds(base, K)], dst_v)
        pltpu.sync_copy(p_hbm.at[pl.ds(base, K)], p_v)
        pltpu.sync_copy(h_hbm.at[src_v], rows_v)

        def row_body(g, _):
            pvec = p_v[pl.ds(g * 16, 16)]
            for l in range(16):
                b = jnp.full((16,), pvec[l], f32)
                r = g * 16 + l
                rows_v[r, pl.ds(0, 16)] = rows_v[r, pl.ds(0, 16)] * b
                rows_v[r, pl.ds(16, 16)] = rows_v[r, pl.ds(16, 16)] * b
            return 0

        lax.fori_loop(0, NVEC, row_body, 0)

        pltpu.sync_copy(rows_v, acc_sh.at[dst_v], add=True)

    plsc.subcore_barrier()

    @pl.when(sid < N // ZR)
    def _():
        pltpu.sync_copy(acc_sh.at[pl.ds(sid * ZR, ZR)], zrow_v)
        pltpu.sync_copy(zrow_v, acc_hbm.at[cid, pl.ds(sid * ZR, ZR)])


def _make_pass2():
    scratch = [
        pltpu.VMEM((K,), i32),
        pltpu.VMEM((K,), i32),
        pltpu.VMEM((K,), f32),
        pltpu.VMEM((K, H), f32),
        pltpu.VMEM((ZR, H), f32),
        pltpu.VMEM_SHARED((N, H), f32),
    ]
    out_type = jax.ShapeDtypeStruct((NC, N, H), f32)
    return pl.kernel(_pass2_body, out_type=out_type, mesh=_MESH,
                     scratch_types=scratch, compiler_params=_SC_PARAMS_NT)


def kernel(x, edge_index, edge_attr, W1, as1, ad1, We1, ae1, b1,
           W2, as2, ad2, We2, ae2, b2, Wl, bl):
    src = edge_index[0]
    dst = edge_index[1]

    ve12 = jnp.stack([We1 @ ae1, We2 @ ae2], axis=1)

    h1, asrc1, adst1, ms1, md1 = _node_dense(x, W1, as1, ad1)
    ae_e1, ae_e2, mae1, mae2 = _edge_dense(edge_attr, ve12)

    bound1 = ms1[0, 0] + md1[0, 0] + jnp.maximum(mae1[0, 0], 0.0)
    bound1_v = jnp.full((16,), bound1, f32)
    bound1_s = bound1.reshape(1, 1)

    p1, accp1, acce1, acce2, accdeg = _make_pass1_l1()(
        src, dst, ae_e1.reshape(E), ae_e2.reshape(E),
        asrc1.reshape(N), adst1.reshape(N), bound1_v)

    acc32_1 = _make_pass2()(src, dst, p1, h1)

    h2, asrc2, adst2, aloop2, ms2, md2 = _mid_dense(
        h1, acc32_1, accp1.reshape(NC, N, 1), acce1.reshape(NC, N, 1),
        acce2.reshape(NC, N, 1), accdeg.reshape(NC, N, 1),
        asrc1, adst1, bound1_s, b1, W2, as2, ad2)

    bound2 = ms2[0, 0] + md2[0, 0] + jnp.maximum(mae2[0, 0], 0.0)
    bound2_v = jnp.full((16,), bound2, f32)
    bound2_s = bound2.reshape(1, 1)

    p2, accp2 = _make_pass1_l2()(
        src, dst, ae_e2.reshape(E),
        asrc2.reshape(N), adst2.reshape(N), bound2_v)

    acc32_2 = _make_pass2()(src, dst, p2, h2)

    out = _final_dense(h2, acc32_2, accp2.reshape(NC, N, 1),
                       asrc2, adst2, aloop2, bound2_s, b2, Wl, bl)
    return out
